# baseline (speedup 1.0000x reference)
import functools

import jax
import jax.numpy as jnp
from jax import lax
from jax.experimental import pallas as pl
from jax.experimental.pallas import tpu as pltpu
from jax.experimental.pallas import tpu_sc as plsc

NN = 10000
NE = 320000
D = 128
NG = 16
CHUNK = 128
NCHUNK = NE // CHUNK
NW = 32
CPW = 80
NCHUNK_PAD = NW * CPW
RPT = 624
HB = 40

def _zero_acc(z_hbm, acc_sp, sid):
    @pl.when(sid < 15)
    def _():
        pltpu.sync_copy(z_hbm.at[pl.ds(0, RPT)],
                        acc_sp.at[pl.ds(sid * RPT, RPT)])

    @pl.when(sid == 15)
    def _():
        pltpu.sync_copy(z_hbm, acc_sp.at[pl.ds(15 * RPT, RPT + 16)])


def _copy_out(acc_sp, acc0_out, acc1_out, cid, sid):
    for c, acc_out in ((0, acc0_out), (1, acc1_out)):
        @pl.when((cid == c) & (sid < 15))
        def _():
            pltpu.sync_copy(acc_sp.at[pl.ds(sid * RPT, RPT)],
                            acc_out.at[pl.ds(sid * RPT, RPT)])

        @pl.when((cid == c) & (sid == 15))
        def _():
            pltpu.sync_copy(acc_sp.at[pl.ds(15 * RPT, RPT + 16)],
                            acc_out.at[pl.ds(15 * RPT, RPT + 16)])


def _sc_agg_body(feat_hbm, src_hbm, dst_hbm, z_hbm, acc0_out, acc1_out,
                 src_v, dst_v, rows_a, rows_b, acc_sp,
                 sem_ga, sem_gb, sem_sa, sem_sb):
    cid = lax.axis_index("c")
    sid = lax.axis_index("s")
    start = (cid * 16 + sid) * CPW

    _zero_acc(z_hbm, acc_sp, sid)
    plsc.subcore_barrier()

    def gth(j, rows, sem):
        return pltpu.make_async_copy(feat_hbm.at[src_v.at[j]], rows, sem)

    def sct(j, rows, sem):
        return pltpu.make_async_copy(rows, acc_sp.at[dst_v.at[j]], sem)

    for h in range(CPW // HB):
        base = start + h * HB

        def act(j):
            return base + j < NCHUNK

        pltpu.sync_copy(src_hbm.at[pl.ds(base, HB)], src_v)
        pltpu.sync_copy(dst_hbm.at[pl.ds(base, HB)], dst_v)

        @pl.when(act(0))
        def _():
            gth(0, rows_a, sem_ga).start()

        def step(jj, carry):
            j0 = 2 * jj
            j1 = 2 * jj + 1

            @pl.when((jj > 0) & act(j1 - 2))
            def _():
                sct(j1 - 2, rows_b, sem_sb).wait()

            @pl.when(act(j1))
            def _():
                gth(j1, rows_b, sem_gb).start()

            @pl.when(act(j0))
            def _():
                gth(j0, rows_a, sem_ga).wait()

            @pl.when(act(j0))
            def _():
                sct(j0, rows_a, sem_sa).start(add=True)

            @pl.when(act(j1))
            def _():
                gth(j1, rows_b, sem_gb).wait()

            @pl.when(act(j1))
            def _():
                sct(j1, rows_b, sem_sb).start(add=True)

            @pl.when(act(j0))
            def _():
                sct(j0, rows_a, sem_sa).wait()

            @pl.when((jj < HB // 2 - 1) & act(j0 + 2))
            def _():
                gth(j0 + 2, rows_a, sem_ga).start()

            return carry

        lax.fori_loop(0, HB // 2, step, 0)

        @pl.when(act(HB - 1))
        def _():
            sct(HB - 1, rows_b, sem_sb).wait()

    plsc.subcore_barrier()
    _copy_out(acc_sp, acc0_out, acc1_out, cid, sid)


def _sc_deg_body(ones_hbm, dst_hbm, z_hbm, deg0_out, deg1_out,
                 dst_v, ones_v, acc_sp, sem):
    cid = lax.axis_index("c")
    sid = lax.axis_index("s")
    start = (cid * 16 + sid) * CPW

    _zero_acc(z_hbm, acc_sp, sid)
    pltpu.sync_copy(ones_hbm, ones_v)
    pltpu.sync_copy(dst_hbm.at[pl.ds(start, CPW)], dst_v)
    plsc.subcore_barrier()

    def step(g, carry):
        for k in range(8):
            @pl.when(start + g * 8 + k < NCHUNK)
            def _():
                pltpu.async_copy(ones_v, acc_sp.at[dst_v.at[g * 8 + k]],
                                 sem, add=True)
        for k in range(8):
            @pl.when(start + g * 8 + k < NCHUNK)
            def _():
                pltpu.make_async_copy(
                    ones_v, acc_sp.at[dst_v.at[g * 8 + k]], sem).wait()
        return carry

    lax.fori_loop(0, CPW // 8, step, 0)
    plsc.subcore_barrier()
    _copy_out(acc_sp, deg0_out, deg1_out, cid, sid)


@functools.cache
def _sc_kernels():
    mesh = plsc.VectorSubcoreMesh(core_axis_name="c", subcore_axis_name="s")
    out2 = [jax.ShapeDtypeStruct((NN, D), jnp.float32),
            jax.ShapeDtypeStruct((NN, D), jnp.float32)]
    agg = pl.kernel(
        _sc_agg_body, mesh=mesh, out_type=out2,
        scratch_types=[
            pltpu.VMEM((HB, CHUNK), jnp.int32),
            pltpu.VMEM((HB, CHUNK), jnp.int32),
            pltpu.VMEM((CHUNK, D), jnp.float32),
            pltpu.VMEM((CHUNK, D), jnp.float32),
            pltpu.VMEM_SHARED((NN, D), jnp.float32),
            pltpu.SemaphoreType.DMA,
            pltpu.SemaphoreType.DMA,
            pltpu.SemaphoreType.DMA,
            pltpu.SemaphoreType.DMA,
        ])
    deg = pl.kernel(
        _sc_deg_body, mesh=mesh, out_type=out2,
        scratch_types=[
            pltpu.VMEM((CPW, CHUNK), jnp.int32),
            pltpu.VMEM((CHUNK, D), jnp.float32),
            pltpu.VMEM_SHARED((NN, D), jnp.float32),
            pltpu.SemaphoreType.DMA,
        ])
    return agg, deg

_CT = (((1,), (1,)), ((), ()))


def _tc_layer_body(p0, p1, dw0, dw1, x, Wl, b, Wr, o, dego):
    deg = dw0[...] + dw1[...]
    dego[...] = deg
    inv = 1.0 / jnp.clip(deg, 1.0, None)
    agg = (p0[...] + p1[...]) * inv
    h = lax.dot_general(agg, Wl[...], _CT, preferred_element_type=jnp.float32)
    h = h + b[...] + lax.dot_general(x[...], Wr[...], _CT,
                                     preferred_element_type=jnp.float32)
    o[...] = jnp.maximum(h, 0.0)


def _tc_final_body(q0, q1, dg, h1, Wl, b, Wr, bat, Wfc, bfc, o,
                   sums, cnts):
    i = pl.program_id(0)

    @pl.when(i == 0)
    def _():
        sums[...] = jnp.zeros((NG, D), jnp.float32)
        cnts[...] = jnp.zeros((NG, D), jnp.float32)

    inv = 1.0 / jnp.clip(dg[...], 1.0, None)
    agg = (q0[...] + q1[...]) * inv
    h = lax.dot_general(agg, Wl[...], _CT, preferred_element_type=jnp.float32)
    h = h + b[...] + lax.dot_general(h1[...], Wr[...], _CT,
                                     preferred_element_type=jnp.float32)
    h2 = jnp.maximum(h, 0.0)

    gid = lax.broadcasted_iota(jnp.int32, (1, NG), 1)
    oh = (bat[...] == gid).astype(jnp.float32)
    ct0 = (((0,), (0,)), ((), ()))
    sums[...] += lax.dot_general(oh, h2, ct0,
                                 preferred_element_type=jnp.float32)
    cnts[...] += lax.dot_general(oh, jnp.ones_like(h2), ct0,
                                 preferred_element_type=jnp.float32)

    @pl.when(i == pl.num_programs(0) - 1)
    def _():
        g = sums[...] / jnp.clip(cnts[...], 1.0, None)
        o[...] = lax.dot_general(g, Wfc[...], _CT,
                                 preferred_element_type=jnp.float32) + bfc[...]


_BLK = 1000


def _tc_layer(p0, p1, dw0, dw1, x, Wl, b, Wr):
    grid = NN // _BLK
    row = pl.BlockSpec((_BLK, D), lambda i: (i, 0))
    bcol = pl.BlockSpec((_BLK, 1), lambda i: (i, 0))
    full = pl.BlockSpec((D, D), lambda i: (0, 0))
    bsp = pl.BlockSpec((1, D), lambda i: (0, 0))
    return pl.pallas_call(
        _tc_layer_body,
        grid=(grid,),
        in_specs=[row, row, bcol, bcol, row, full, bsp, full],
        out_specs=[row, bcol],
        out_shape=[jax.ShapeDtypeStruct((NN, D), jnp.float32),
                   jax.ShapeDtypeStruct((NN, 1), jnp.float32)],
    )(p0, p1, dw0, dw1, x, Wl, b, Wr)


def _tc_final(q0, q1, deg, h1, Wl, b, Wr, bat, Wfc, bfc):
    grid = NN // _BLK
    row = pl.BlockSpec((_BLK, D), lambda i: (i, 0))
    bcol = pl.BlockSpec((_BLK, 1), lambda i: (i, 0))
    full = pl.BlockSpec((D, D), lambda i: (0, 0))
    bsp = pl.BlockSpec((1, D), lambda i: (0, 0))
    wfc = pl.BlockSpec((64, D), lambda i: (0, 0))
    bfcs = pl.BlockSpec((1, 64), lambda i: (0, 0))
    osp = pl.BlockSpec((NG, 64), lambda i: (0, 0))
    return pl.pallas_call(
        _tc_final_body,
        grid=(grid,),
        in_specs=[row, row, bcol, row, full, bsp, full, bcol, wfc,
                  bfcs],
        out_specs=osp,
        out_shape=jax.ShapeDtypeStruct((NG, 64), jnp.float32),
        scratch_shapes=[pltpu.VMEM((NG, D), jnp.float32),
                        pltpu.VMEM((NG, D), jnp.float32)],
    )(q0, q1, deg, h1, Wl, b, Wr, bat, Wfc, bfc)


def kernel(x, edge_index, batch, W1l, b1l, W1r, W2l, b2l, W2r, Wfc, bfc):
    src2 = jnp.pad(edge_index[0].astype(jnp.int32).reshape(NCHUNK, CHUNK),
                   ((0, NCHUNK_PAD - NCHUNK), (0, 0)))
    dst2 = jnp.pad(edge_index[1].astype(jnp.int32).reshape(NCHUNK, CHUNK),
                   ((0, NCHUNK_PAD - NCHUNK), (0, 0)))
    z = jnp.zeros((RPT + 16, D), jnp.float32)
    ones = jnp.ones((CHUNK, D), jnp.float32)

    sc_agg, sc_deg = _sc_kernels()
    dw0, dw1 = sc_deg(ones, dst2, z)
    p0, p1 = sc_agg(x, src2, dst2, z)
    h1, deg = _tc_layer(p0, p1, dw0[:, :1], dw1[:, :1], x, W1l,
                        b1l.reshape(1, D), W1r)
    q0, q1 = sc_agg(h1, src2, dst2, z)
    out = _tc_final(q0, q1, deg, h1, W2l, b2l.reshape(1, D), W2r,
                    batch.astype(jnp.int32).reshape(NN, 1), Wfc,
                    bfc.reshape(1, 64))
    return out


# speedup vs baseline: 1.0758x; 1.0758x over previous
import functools

import jax
import jax.numpy as jnp
from jax import lax
from jax.experimental import pallas as pl
from jax.experimental.pallas import tpu as pltpu
from jax.experimental.pallas import tpu_sc as plsc

NN = 10000
NE = 320000
D = 128
NG = 16
CHUNK = 128
NCHUNK = NE // CHUNK
NW = 32
CPW = 80
NCHUNK_PAD = NW * CPW
RPT = 624
HB = 40

def _zero_acc(z_hbm, acc_sp, sid):
    @pl.when(sid < 15)
    def _():
        pltpu.sync_copy(z_hbm.at[pl.ds(0, RPT)],
                        acc_sp.at[pl.ds(sid * RPT, RPT)])

    @pl.when(sid == 15)
    def _():
        pltpu.sync_copy(z_hbm, acc_sp.at[pl.ds(15 * RPT, RPT + 16)])


def _copy_out(acc_sp, acc0_out, acc1_out, cid, sid):
    for c, acc_out in ((0, acc0_out), (1, acc1_out)):
        @pl.when((cid == c) & (sid < 15))
        def _():
            pltpu.sync_copy(acc_sp.at[pl.ds(sid * RPT, RPT)],
                            acc_out.at[pl.ds(sid * RPT, RPT)])

        @pl.when((cid == c) & (sid == 15))
        def _():
            pltpu.sync_copy(acc_sp.at[pl.ds(15 * RPT, RPT + 16)],
                            acc_out.at[pl.ds(15 * RPT, RPT + 16)])


def _sc_agg_body(feat_hbm, src_hbm, dst_hbm, z_hbm, acc0_out, acc1_out,
                 src_v, dst_v, rows_a, rows_b, acc_sp,
                 sem_ga, sem_gb, sem_sa, sem_sb):
    cid = lax.axis_index("c")
    sid = lax.axis_index("s")
    start = (cid * 16 + sid) * CPW

    _zero_acc(z_hbm, acc_sp, sid)
    plsc.subcore_barrier()

    def gth(j, rows, sem):
        return pltpu.make_async_copy(feat_hbm.at[src_v.at[j]], rows, sem)

    def sct(j, rows, sem):
        return pltpu.make_async_copy(rows, acc_sp.at[dst_v.at[j]], sem)

    for h in range(CPW // HB):
        base = start + h * HB

        def act(j):
            return base + j < NCHUNK

        pltpu.sync_copy(src_hbm.at[pl.ds(base, HB)], src_v)
        pltpu.sync_copy(dst_hbm.at[pl.ds(base, HB)], dst_v)

        @pl.when(act(0))
        def _():
            gth(0, rows_a, sem_ga).start()

        def step(jj, carry):
            j0 = 2 * jj
            j1 = 2 * jj + 1

            @pl.when(act(j0))
            def _():
                gth(j0, rows_a, sem_ga).wait()

            @pl.when(act(j0))
            def _():
                sct(j0, rows_a, sem_sa).start(add=True)

            @pl.when((jj > 0) & act(j1 - 2))
            def _():
                sct(j1 - 2, rows_b, sem_sb).wait()

            @pl.when(act(j1))
            def _():
                gth(j1, rows_b, sem_gb).start()

            @pl.when(act(j1))
            def _():
                gth(j1, rows_b, sem_gb).wait()

            @pl.when(act(j0))
            def _():
                sct(j0, rows_a, sem_sa).wait()

            @pl.when((jj < HB // 2 - 1) & act(j0 + 2))
            def _():
                gth(j0 + 2, rows_a, sem_ga).start()

            @pl.when(act(j1))
            def _():
                sct(j1, rows_b, sem_sb).start(add=True)

            return carry

        lax.fori_loop(0, HB // 2, step, 0)

        @pl.when(act(HB - 1))
        def _():
            sct(HB - 1, rows_b, sem_sb).wait()

    plsc.subcore_barrier()
    _copy_out(acc_sp, acc0_out, acc1_out, cid, sid)


def _sc_deg_body(ones_hbm, dst_hbm, z_hbm, deg0_out, deg1_out,
                 dst_v, ones_v, acc_sp, sem):
    cid = lax.axis_index("c")
    sid = lax.axis_index("s")
    start = (cid * 16 + sid) * CPW

    _zero_acc(z_hbm, acc_sp, sid)
    pltpu.sync_copy(ones_hbm, ones_v)
    pltpu.sync_copy(dst_hbm.at[pl.ds(start, CPW)], dst_v)
    plsc.subcore_barrier()

    def step(g, carry):
        for k in range(8):
            @pl.when(start + g * 8 + k < NCHUNK)
            def _():
                pltpu.async_copy(ones_v, acc_sp.at[dst_v.at[g * 8 + k]],
                                 sem, add=True)
        for k in range(8):
            @pl.when(start + g * 8 + k < NCHUNK)
            def _():
                pltpu.make_async_copy(
                    ones_v, acc_sp.at[dst_v.at[g * 8 + k]], sem).wait()
        return carry

    lax.fori_loop(0, CPW // 8, step, 0)
    plsc.subcore_barrier()
    _copy_out(acc_sp, deg0_out, deg1_out, cid, sid)


@functools.cache
def _sc_kernels():
    mesh = plsc.VectorSubcoreMesh(core_axis_name="c", subcore_axis_name="s")
    out2 = [jax.ShapeDtypeStruct((NN, D), jnp.float32),
            jax.ShapeDtypeStruct((NN, D), jnp.float32)]
    agg = pl.kernel(
        _sc_agg_body, mesh=mesh, out_type=out2,
        scratch_types=[
            pltpu.VMEM((HB, CHUNK), jnp.int32),
            pltpu.VMEM((HB, CHUNK), jnp.int32),
            pltpu.VMEM((CHUNK, D), jnp.float32),
            pltpu.VMEM((CHUNK, D), jnp.float32),
            pltpu.VMEM_SHARED((NN, D), jnp.float32),
            pltpu.SemaphoreType.DMA,
            pltpu.SemaphoreType.DMA,
            pltpu.SemaphoreType.DMA,
            pltpu.SemaphoreType.DMA,
        ])
    deg = pl.kernel(
        _sc_deg_body, mesh=mesh, out_type=out2,
        scratch_types=[
            pltpu.VMEM((CPW, CHUNK), jnp.int32),
            pltpu.VMEM((CHUNK, D), jnp.float32),
            pltpu.VMEM_SHARED((NN, D), jnp.float32),
            pltpu.SemaphoreType.DMA,
        ])
    return agg, deg

_CT = (((1,), (1,)), ((), ()))


def _tc_layer_body(p0, p1, dw0, dw1, x, Wl, b, Wr, o, dego):
    deg = dw0[...] + dw1[...]
    dego[...] = deg
    inv = 1.0 / jnp.clip(deg, 1.0, None)
    agg = (p0[...] + p1[...]) * inv
    h = lax.dot_general(agg, Wl[...], _CT, preferred_element_type=jnp.float32)
    h = h + b[...] + lax.dot_general(x[...], Wr[...], _CT,
                                     preferred_element_type=jnp.float32)
    o[...] = jnp.maximum(h, 0.0)


def _tc_final_body(q0, q1, dg, h1, Wl, b, Wr, bat, Wfc, bfc, o,
                   sums, cnts):
    i = pl.program_id(0)

    @pl.when(i == 0)
    def _():
        sums[...] = jnp.zeros((NG, D), jnp.float32)
        cnts[...] = jnp.zeros((NG, D), jnp.float32)

    inv = 1.0 / jnp.clip(dg[...], 1.0, None)
    agg = (q0[...] + q1[...]) * inv
    h = lax.dot_general(agg, Wl[...], _CT, preferred_element_type=jnp.float32)
    h = h + b[...] + lax.dot_general(h1[...], Wr[...], _CT,
                                     preferred_element_type=jnp.float32)
    h2 = jnp.maximum(h, 0.0)

    gid = lax.broadcasted_iota(jnp.int32, (1, NG), 1)
    oh = (bat[...] == gid).astype(jnp.float32)
    ct0 = (((0,), (0,)), ((), ()))
    sums[...] += lax.dot_general(oh, h2, ct0,
                                 preferred_element_type=jnp.float32)
    cnts[...] += lax.dot_general(oh, jnp.ones_like(h2), ct0,
                                 preferred_element_type=jnp.float32)

    @pl.when(i == pl.num_programs(0) - 1)
    def _():
        g = sums[...] / jnp.clip(cnts[...], 1.0, None)
        o[...] = lax.dot_general(g, Wfc[...], _CT,
                                 preferred_element_type=jnp.float32) + bfc[...]


_BLK = 1000


def _tc_layer(p0, p1, dw0, dw1, x, Wl, b, Wr):
    grid = NN // _BLK
    row = pl.BlockSpec((_BLK, D), lambda i: (i, 0))
    bcol = pl.BlockSpec((_BLK, 1), lambda i: (i, 0))
    full = pl.BlockSpec((D, D), lambda i: (0, 0))
    bsp = pl.BlockSpec((1, D), lambda i: (0, 0))
    return pl.pallas_call(
        _tc_layer_body,
        grid=(grid,),
        in_specs=[row, row, bcol, bcol, row, full, bsp, full],
        out_specs=[row, bcol],
        out_shape=[jax.ShapeDtypeStruct((NN, D), jnp.float32),
                   jax.ShapeDtypeStruct((NN, 1), jnp.float32)],
    )(p0, p1, dw0, dw1, x, Wl, b, Wr)


def _tc_final(q0, q1, deg, h1, Wl, b, Wr, bat, Wfc, bfc):
    grid = NN // _BLK
    row = pl.BlockSpec((_BLK, D), lambda i: (i, 0))
    bcol = pl.BlockSpec((_BLK, 1), lambda i: (i, 0))
    full = pl.BlockSpec((D, D), lambda i: (0, 0))
    bsp = pl.BlockSpec((1, D), lambda i: (0, 0))
    wfc = pl.BlockSpec((64, D), lambda i: (0, 0))
    bfcs = pl.BlockSpec((1, 64), lambda i: (0, 0))
    osp = pl.BlockSpec((NG, 64), lambda i: (0, 0))
    return pl.pallas_call(
        _tc_final_body,
        grid=(grid,),
        in_specs=[row, row, bcol, row, full, bsp, full, bcol, wfc,
                  bfcs],
        out_specs=osp,
        out_shape=jax.ShapeDtypeStruct((NG, 64), jnp.float32),
        scratch_shapes=[pltpu.VMEM((NG, D), jnp.float32),
                        pltpu.VMEM((NG, D), jnp.float32)],
    )(q0, q1, deg, h1, Wl, b, Wr, bat, Wfc, bfc)


def kernel(x, edge_index, batch, W1l, b1l, W1r, W2l, b2l, W2r, Wfc, bfc):
    src2 = jnp.pad(edge_index[0].astype(jnp.int32).reshape(NCHUNK, CHUNK),
                   ((0, NCHUNK_PAD - NCHUNK), (0, 0)))
    dst2 = jnp.pad(edge_index[1].astype(jnp.int32).reshape(NCHUNK, CHUNK),
                   ((0, NCHUNK_PAD - NCHUNK), (0, 0)))
    z = jnp.zeros((RPT + 16, D), jnp.float32)
    ones = jnp.ones((CHUNK, D), jnp.float32)

    sc_agg, sc_deg = _sc_kernels()
    dw0, dw1 = sc_deg(ones, dst2, z)
    p0, p1 = sc_agg(x, src2, dst2, z)
    h1, deg = _tc_layer(p0, p1, dw0[:, :1], dw1[:, :1], x, W1l,
                        b1l.reshape(1, D), W1r)
    q0, q1 = sc_agg(h1, src2, dst2, z)
    out = _tc_final(q0, q1, deg, h1, W2l, b2l.reshape(1, D), W2r,
                    batch.astype(jnp.int32).reshape(NN, 1), Wfc,
                    bfc.reshape(1, 64))
    return out


# speedup vs baseline: 1.0859x; 1.0094x over previous
import functools

import jax
import jax.numpy as jnp
from jax import lax
from jax.experimental import pallas as pl
from jax.experimental.pallas import tpu as pltpu
from jax.experimental.pallas import tpu_sc as plsc

NN = 10000
NE = 320000
D = 128
NG = 16
CHUNK = 128
NCHUNK = NE // CHUNK
NW = 32
CPW = 80
NCHUNK_PAD = NW * CPW
RPT = 624
HB = 40

def _zero_acc(z_hbm, acc_sp, sid):
    @pl.when(sid < 15)
    def _():
        pltpu.sync_copy(z_hbm.at[pl.ds(0, RPT)],
                        acc_sp.at[pl.ds(sid * RPT, RPT)])

    @pl.when(sid == 15)
    def _():
        pltpu.sync_copy(z_hbm, acc_sp.at[pl.ds(15 * RPT, RPT + 16)])


def _copy_out(acc_sp, acc0_out, acc1_out, cid, sid):
    for c, acc_out in ((0, acc0_out), (1, acc1_out)):
        @pl.when((cid == c) & (sid < 15))
        def _():
            pltpu.sync_copy(acc_sp.at[pl.ds(sid * RPT, RPT)],
                            acc_out.at[pl.ds(sid * RPT, RPT)])

        @pl.when((cid == c) & (sid == 15))
        def _():
            pltpu.sync_copy(acc_sp.at[pl.ds(15 * RPT, RPT + 16)],
                            acc_out.at[pl.ds(15 * RPT, RPT + 16)])


def _sc_agg_body(feat_hbm, src_hbm, dst_hbm, z_hbm, acc0_out, acc1_out,
                 src_v, dst_v, rows_a, rows_b, acc_sp,
                 sem_ga, sem_gb, sem_sa, sem_sb,
                 ones_hbm=None, deg0_out=None, deg1_out=None):
    cid = lax.axis_index("c")
    sid = lax.axis_index("s")
    start = (cid * 16 + sid) * CPW

    if ones_hbm is not None:
        _zero_acc(z_hbm, acc_sp, sid)
        pltpu.sync_copy(ones_hbm, rows_a)
        plsc.subcore_barrier()
        for h in range(CPW // HB):
            dbase = start + h * HB
            pltpu.sync_copy(dst_hbm.at[pl.ds(dbase, HB)], dst_v)

            def dstep(g, carry):
                for k in range(8):
                    @pl.when(dbase + g * 8 + k < NCHUNK)
                    def _():
                        pltpu.async_copy(
                            rows_a, acc_sp.at[dst_v.at[g * 8 + k]],
                            sem_sa, add=True)
                for k in range(8):
                    @pl.when(dbase + g * 8 + k < NCHUNK)
                    def _():
                        pltpu.make_async_copy(
                            rows_a, acc_sp.at[dst_v.at[g * 8 + k]],
                            sem_sa).wait()
                return carry

            lax.fori_loop(0, HB // 8, dstep, 0)
        plsc.subcore_barrier()
        _copy_out(acc_sp, deg0_out, deg1_out, cid, sid)
        plsc.subcore_barrier()

    _zero_acc(z_hbm, acc_sp, sid)
    plsc.subcore_barrier()

    def gth(j, rows, sem):
        return pltpu.make_async_copy(feat_hbm.at[src_v.at[j]], rows, sem)

    def sct(j, rows, sem):
        return pltpu.make_async_copy(rows, acc_sp.at[dst_v.at[j]], sem)

    for h in range(CPW // HB):
        base = start + h * HB

        def act(j):
            return base + j < NCHUNK

        pltpu.sync_copy(src_hbm.at[pl.ds(base, HB)], src_v)
        pltpu.sync_copy(dst_hbm.at[pl.ds(base, HB)], dst_v)

        @pl.when(act(0))
        def _():
            gth(0, rows_a, sem_ga).start()

        def step(jj, carry):
            j0 = 2 * jj
            j1 = 2 * jj + 1

            @pl.when(act(j0))
            def _():
                gth(j0, rows_a, sem_ga).wait()

            @pl.when(act(j0))
            def _():
                sct(j0, rows_a, sem_sa).start(add=True)

            @pl.when((jj > 0) & act(j1 - 2))
            def _():
                sct(j1 - 2, rows_b, sem_sb).wait()

            @pl.when(act(j1))
            def _():
                gth(j1, rows_b, sem_gb).start()

            @pl.when(act(j1))
            def _():
                gth(j1, rows_b, sem_gb).wait()

            @pl.when(act(j0))
            def _():
                sct(j0, rows_a, sem_sa).wait()

            @pl.when((jj < HB // 2 - 1) & act(j0 + 2))
            def _():
                gth(j0 + 2, rows_a, sem_ga).start()

            @pl.when(act(j1))
            def _():
                sct(j1, rows_b, sem_sb).start(add=True)

            return carry

        lax.fori_loop(0, HB // 2, step, 0)

        @pl.when(act(HB - 1))
        def _():
            sct(HB - 1, rows_b, sem_sb).wait()

    plsc.subcore_barrier()
    _copy_out(acc_sp, acc0_out, acc1_out, cid, sid)


_SCRATCH = [
    pltpu.VMEM((HB, CHUNK), jnp.int32),
    pltpu.VMEM((HB, CHUNK), jnp.int32),
    pltpu.VMEM((CHUNK, D), jnp.float32),
    pltpu.VMEM((CHUNK, D), jnp.float32),
    pltpu.VMEM_SHARED((NN, D), jnp.float32),
    pltpu.SemaphoreType.DMA,
    pltpu.SemaphoreType.DMA,
    pltpu.SemaphoreType.DMA,
    pltpu.SemaphoreType.DMA,
]


def _fused_body(feat, src, dst, z, ones, a0, a1, g0, g1, *scr):
    _sc_agg_body(feat, src, dst, z, a0, a1, *scr,
                 ones_hbm=ones, deg0_out=g0, deg1_out=g1)


@functools.cache
def _sc_kernels():
    mesh = plsc.VectorSubcoreMesh(core_axis_name="c", subcore_axis_name="s")
    out2 = [jax.ShapeDtypeStruct((NN, D), jnp.float32)] * 2
    agg = pl.kernel(_sc_agg_body, mesh=mesh, out_type=out2,
                    scratch_types=list(_SCRATCH))
    agg_deg = pl.kernel(_fused_body, mesh=mesh, out_type=out2 * 2,
                        scratch_types=list(_SCRATCH))
    return agg, agg_deg

_CT = (((1,), (1,)), ((), ()))


def _tc_layer_body(p0, p1, dw0, dw1, x, Wl, b, Wr, o, dego):
    deg = dw0[...] + dw1[...]
    dego[...] = deg
    inv = 1.0 / jnp.clip(deg, 1.0, None)
    agg = (p0[...] + p1[...]) * inv
    h = lax.dot_general(agg, Wl[...], _CT, preferred_element_type=jnp.float32)
    h = h + b[...] + lax.dot_general(x[...], Wr[...], _CT,
                                     preferred_element_type=jnp.float32)
    o[...] = jnp.maximum(h, 0.0)


def _tc_final_body(q0, q1, dg, h1, Wl, b, Wr, bat, Wfc, bfc, o,
                   sums, cnts):
    i = pl.program_id(0)

    @pl.when(i == 0)
    def _():
        sums[...] = jnp.zeros((NG, D), jnp.float32)
        cnts[...] = jnp.zeros((NG, D), jnp.float32)

    inv = 1.0 / jnp.clip(dg[...], 1.0, None)
    agg = (q0[...] + q1[...]) * inv
    h = lax.dot_general(agg, Wl[...], _CT, preferred_element_type=jnp.float32)
    h = h + b[...] + lax.dot_general(h1[...], Wr[...], _CT,
                                     preferred_element_type=jnp.float32)
    h2 = jnp.maximum(h, 0.0)

    gid = lax.broadcasted_iota(jnp.int32, (1, NG), 1)
    oh = (bat[...] == gid).astype(jnp.float32)
    ct0 = (((0,), (0,)), ((), ()))
    sums[...] += lax.dot_general(oh, h2, ct0,
                                 preferred_element_type=jnp.float32)
    cnts[...] += lax.dot_general(oh, jnp.ones_like(h2), ct0,
                                 preferred_element_type=jnp.float32)

    @pl.when(i == pl.num_programs(0) - 1)
    def _():
        g = sums[...] / jnp.clip(cnts[...], 1.0, None)
        o[...] = lax.dot_general(g, Wfc[...], _CT,
                                 preferred_element_type=jnp.float32) + bfc[...]


_BLK = 1000


def _tc_layer(p0, p1, dw0, dw1, x, Wl, b, Wr):
    grid = NN // _BLK
    row = pl.BlockSpec((_BLK, D), lambda i: (i, 0))
    bcol = pl.BlockSpec((_BLK, 1), lambda i: (i, 0))
    full = pl.BlockSpec((D, D), lambda i: (0, 0))
    bsp = pl.BlockSpec((1, D), lambda i: (0, 0))
    return pl.pallas_call(
        _tc_layer_body,
        grid=(grid,),
        in_specs=[row, row, bcol, bcol, row, full, bsp, full],
        out_specs=[row, bcol],
        out_shape=[jax.ShapeDtypeStruct((NN, D), jnp.float32),
                   jax.ShapeDtypeStruct((NN, 1), jnp.float32)],
    )(p0, p1, dw0, dw1, x, Wl, b, Wr)


def _tc_final(q0, q1, deg, h1, Wl, b, Wr, bat, Wfc, bfc):
    grid = NN // _BLK
    row = pl.BlockSpec((_BLK, D), lambda i: (i, 0))
    bcol = pl.BlockSpec((_BLK, 1), lambda i: (i, 0))
    full = pl.BlockSpec((D, D), lambda i: (0, 0))
    bsp = pl.BlockSpec((1, D), lambda i: (0, 0))
    wfc = pl.BlockSpec((64, D), lambda i: (0, 0))
    bfcs = pl.BlockSpec((1, 64), lambda i: (0, 0))
    osp = pl.BlockSpec((NG, 64), lambda i: (0, 0))
    return pl.pallas_call(
        _tc_final_body,
        grid=(grid,),
        in_specs=[row, row, bcol, row, full, bsp, full, bcol, wfc,
                  bfcs],
        out_specs=osp,
        out_shape=jax.ShapeDtypeStruct((NG, 64), jnp.float32),
        scratch_shapes=[pltpu.VMEM((NG, D), jnp.float32),
                        pltpu.VMEM((NG, D), jnp.float32)],
    )(q0, q1, deg, h1, Wl, b, Wr, bat, Wfc, bfc)


def kernel(x, edge_index, batch, W1l, b1l, W1r, W2l, b2l, W2r, Wfc, bfc):
    src2 = jnp.pad(edge_index[0].astype(jnp.int32).reshape(NCHUNK, CHUNK),
                   ((0, NCHUNK_PAD - NCHUNK), (0, 0)))
    dst2 = jnp.pad(edge_index[1].astype(jnp.int32).reshape(NCHUNK, CHUNK),
                   ((0, NCHUNK_PAD - NCHUNK), (0, 0)))
    z = jnp.zeros((RPT + 16, D), jnp.float32)
    ones = jnp.ones((CHUNK, D), jnp.float32)

    sc_agg, sc_agg_deg = _sc_kernels()
    p0, p1, dw0, dw1 = sc_agg_deg(x, src2, dst2, z, ones)
    h1, deg = _tc_layer(p0, p1, dw0[:, :1], dw1[:, :1], x, W1l,
                        b1l.reshape(1, D), W1r)
    q0, q1 = sc_agg(h1, src2, dst2, z)
    out = _tc_final(q0, q1, deg, h1, W2l, b2l.reshape(1, D), W2r,
                    batch.astype(jnp.int32).reshape(NN, 1), Wfc,
                    bfc.reshape(1, 64))
    return out
